# baseline (speedup 1.0000x reference)
import functools

import jax
import jax.numpy as jnp
from jax import lax
from jax.experimental import pallas as pl
from jax.experimental.pallas import tpu as pltpu
from jax.experimental.pallas import tpu_sc as plsc

_Z = 1.0
_NEWTON_ITERS = 12
_L = 16


def _sc_body(x_hbm, wp_hbm, wc_hbm, xbuf, wpbuf, cand):
    n = x_hbm.shape[-1]
    nchunks = n // _L
    wid = lax.axis_index("s") * 2 + lax.axis_index("c")

    for r in range(4):
        row = wid * 4 + r
        pltpu.sync_copy(x_hbm.at[row], xbuf)

        def p1(c, m):
            return jnp.maximum(m, xbuf[pl.ds(c * _L, _L)])

        mvec = lax.fori_loop(0, nchunks, p1, jnp.full((_L,), -jnp.inf, jnp.float32))
        tau0 = jnp.full((_L,), jnp.max(mvec) - _Z, jnp.float32)

        def p2(c, off):
            v = xbuf[pl.ds(c * _L, _L)]
            msk = v > tau0
            plsc.store_compressed(cand.at[pl.ds(off, _L)], v, mask=msk)
            cnt = plsc.all_reduce_population_count(msk)
            return off + jnp.max(cnt)

        ncand = lax.fori_loop(0, nchunks, p2, jnp.int32(0))
        ncchunks = (ncand + _L - 1) // _L

        zero_v = jnp.zeros((_L,), jnp.float32)

        def nstep(_, tau):
            def inner(c, acc):
                sv, kv = acc
                v = cand[pl.ds(c * _L, _L)]
                idx = c * _L + lax.iota(jnp.int32, _L)
                act = (idx < ncand) & (v > tau)
                sv = sv + jnp.where(act, v, 0.0)
                kv = kv + jnp.where(act, 1.0, 0.0)
                return (sv, kv)

            sv, kv = lax.fori_loop(0, ncchunks, inner, (zero_v, zero_v))
            s = jnp.full((_L,), jnp.sum(sv), jnp.float32)
            k = jnp.full((_L,), jnp.sum(kv), jnp.float32)
            return (s - _Z) / k

        tau = lax.fori_loop(0, _NEWTON_ITERS, nstep, tau0)

        def p3(c, _):
            v = xbuf[pl.ds(c * _L, _L)]
            wp = jnp.maximum(v - tau, 0.0)
            wpbuf[pl.ds(c * _L, _L)] = wp
            xbuf[pl.ds(c * _L, _L)] = v - wp
            return 0

        lax.fori_loop(0, nchunks, p3, 0)
        pltpu.sync_copy(wpbuf, wp_hbm.at[row])
        pltpu.sync_copy(xbuf, wc_hbm.at[row])


def kernel(x):
    b, n = x.shape
    mesh = plsc.VectorSubcoreMesh(core_axis_name="c", subcore_axis_name="s")
    out = jax.ShapeDtypeStruct((b, n), jnp.float32)
    f = pl.kernel(
        _sc_body,
        out_type=(out, out),
        mesh=mesh,
        scratch_types=[
            pltpu.VMEM((n,), jnp.float32),
            pltpu.VMEM((n,), jnp.float32),
            pltpu.VMEM((n + _L,), jnp.float32),
        ],
        compiler_params=pltpu.CompilerParams(needs_layout_passes=False),
    )
    return f(x)


# speedup vs baseline: 1.3704x; 1.3704x over previous
import jax
import jax.numpy as jnp
from jax import lax
from jax.experimental import pallas as pl
from jax.experimental.pallas import tpu as pltpu
from jax.experimental.pallas import tpu_sc as plsc

_Z = 1.0
_NEWTON_ITERS = 12
_L = 16
_U = 8


def _sc_body(x_hbm, wp_hbm, wc_hbm, xbuf, wpbuf, cand):
    n = x_hbm.shape[-1]
    nchunks = n // _L
    wid = lax.axis_index("s") * 2 + lax.axis_index("c")

    for r in range(4):
        row = wid * 4 + r
        pltpu.sync_copy(x_hbm.at[row], xbuf)

        def p1(g, m):
            base = g * (_U * _L)
            for u in range(_U):
                m = jnp.maximum(m, xbuf[pl.ds(base + u * _L, _L)])
            return m

        mvec = lax.fori_loop(
            0, nchunks // _U, p1, jnp.full((_L,), -jnp.inf, jnp.float32)
        )
        tau0 = jnp.full((_L,), jnp.max(mvec) - _Z, jnp.float32)

        def p2(g, off):
            base = g * (_U * _L)
            for u in range(_U):
                v = xbuf[pl.ds(base + u * _L, _L)]
                msk = v > tau0
                pos = plsc.cumsum(jnp.where(msk, 1, 0).astype(jnp.int32))
                plsc.store_scatter(cand, [off + pos - 1], v, mask=msk)
                off = off + plsc.all_reduce_population_count(msk)
            return off

        offv = lax.fori_loop(0, nchunks // _U, p2, jnp.zeros((_L,), jnp.int32))
        ncand = jnp.max(offv)
        ncchunks = (ncand + _L - 1) // _L

        zero_v = jnp.zeros((_L,), jnp.float32)

        def nstep(_, tau):
            def inner(c, acc):
                sv, kv = acc
                v = cand[pl.ds(c * _L, _L)]
                idx = c * _L + lax.iota(jnp.int32, _L)
                act = (idx < ncand) & (v > tau)
                sv = sv + jnp.where(act, v, 0.0)
                kv = kv + jnp.where(act, 1.0, 0.0)
                return (sv, kv)

            sv, kv = lax.fori_loop(0, ncchunks, inner, (zero_v, zero_v))
            s = jnp.full((_L,), jnp.sum(sv), jnp.float32)
            k = jnp.full((_L,), jnp.sum(kv), jnp.float32)
            return (s - _Z) / k

        tau = lax.fori_loop(0, _NEWTON_ITERS, nstep, tau0)

        def p3(g, _):
            base = g * (_U * _L)
            for u in range(_U):
                sl = pl.ds(base + u * _L, _L)
                v = xbuf[sl]
                wp = jnp.maximum(v - tau, 0.0)
                wpbuf[sl] = wp
                xbuf[sl] = v - wp
            return 0

        lax.fori_loop(0, nchunks // _U, p3, 0)
        pltpu.sync_copy(wpbuf, wp_hbm.at[row])
        pltpu.sync_copy(xbuf, wc_hbm.at[row])


def kernel(x):
    b, n = x.shape
    mesh = plsc.VectorSubcoreMesh(core_axis_name="c", subcore_axis_name="s")
    out = jax.ShapeDtypeStruct((b, n), jnp.float32)
    f = pl.kernel(
        _sc_body,
        out_type=(out, out),
        mesh=mesh,
        scratch_types=[
            pltpu.VMEM((n,), jnp.float32),
            pltpu.VMEM((n,), jnp.float32),
            pltpu.VMEM((n + _L,), jnp.float32),
        ],
        compiler_params=pltpu.CompilerParams(needs_layout_passes=False),
    )
    return f(x)
